# baseline (speedup 1.0000x reference)
import functools

import jax
import jax.numpy as jnp
from jax import lax
from jax.experimental import pallas as pl
from jax.experimental.pallas import tpu as pltpu
from jax.experimental.pallas import tpu_sc as plsc

NUM_FEATS = 128
H = 200
W = 200
NC = 2
NS = 16
LANES = 16
NWORKERS = NC * NS
C_TOTAL = 2 * NUM_FEATS
C_PER_W = C_TOTAL // NWORKERS


def _make_kernel(bs):
    mesh = plsc.VectorSubcoreMesh(
        core_axis_name="c", subcore_axis_name="s",
        num_cores=NC, num_subcores=NS)

    n_full = W // LANES
    tail_off = W - LANES

    @functools.partial(
        pl.kernel,
        out_type=jax.ShapeDtypeStruct((bs, C_TOTAL, H, W), jnp.float32),
        mesh=mesh,
        scratch_types=[
            pltpu.VMEM((C_PER_W, H), jnp.float32),
            pltpu.VMEM((H, W), jnp.float32),
            pltpu.VMEM((H, W), jnp.float32),
            pltpu.SemaphoreType.DMA,
            pltpu.SemaphoreType.DMA,
        ],
        compiler_params=pltpu.CompilerParams(
            use_tc_tiling_on_sc=False, needs_layout_passes=False),
    )
    def body(t_hbm, out_hbm, tv, buf0, buf1, sem0, sem1):
        wid = lax.axis_index("s") * NC + lax.axis_index("c")
        base_c = wid * C_PER_W
        is_col = wid < NWORKERS // 2

        pltpu.sync_copy(t_hbm.at[pl.ds(base_c, C_PER_W)], tv)

        bufs = (buf0, buf1)
        sems = (sem0, sem1)
        pending = {0: [], 1: []}

        for ci in range(C_PER_W):
            k = ci % 2
            buf = bufs[k]
            for hcopy in pending[k]:
                hcopy.wait()
            pending[k] = []

            @pl.when(is_col)
            def _(buf=buf, ci=ci):
                vs = [tv[ci, pl.ds(j * LANES, LANES)] for j in range(n_full)]
                vtail = tv[ci, pl.ds(tail_off, LANES)]

                def row_body(y, carry):
                    for j in range(n_full):
                        buf[y, pl.ds(j * LANES, LANES)] = vs[j]
                    buf[y, pl.ds(tail_off, LANES)] = vtail
                    return carry

                lax.fori_loop(0, H, row_body, 0)

            @pl.when(jnp.logical_not(is_col))
            def _(buf=buf, ci=ci):
                row_idx = jnp.full((LANES,), ci, jnp.int32)

                def row_body(y, carry):
                    v = plsc.load_gather(
                        tv, [row_idx, jnp.full((LANES,), y, jnp.int32)])
                    for j in range(n_full):
                        buf[y, pl.ds(j * LANES, LANES)] = v
                    buf[y, pl.ds(tail_off, LANES)] = v
                    return carry

                lax.fori_loop(0, H, row_body, 0)

            cch = base_c + ci
            for b in range(bs):
                pending[k].append(
                    pltpu.async_copy(buf, out_hbm.at[b, cch], sems[k]))

        for k in (0, 1):
            for hcopy in pending[k]:
                hcopy.wait()

    return body


def kernel(mask, row_weight, col_weight):
    bs = mask.shape[0]
    t = jnp.concatenate([col_weight.T, row_weight.T], axis=0)
    return _make_kernel(bs)(t)


# speedup vs baseline: 1.7323x; 1.7323x over previous
import functools

import jax
import jax.numpy as jnp
from jax import lax
from jax.experimental import pallas as pl
from jax.experimental.pallas import tpu as pltpu
from jax.experimental.pallas import tpu_sc as plsc

NUM_FEATS = 128
H = 200
W = 200
NC = 2
NS = 16
LANES = 16
NWORKERS = NC * NS
C_TOTAL = 2 * NUM_FEATS
C_PER_W = C_TOTAL // NWORKERS


def _make_kernel(bs):
    mesh = plsc.VectorSubcoreMesh(
        core_axis_name="c", subcore_axis_name="s",
        num_cores=NC, num_subcores=NS)

    n_full = W // LANES
    tail_off = W - LANES

    @functools.partial(
        pl.kernel,
        out_type=jax.ShapeDtypeStruct((bs, C_TOTAL, H * W), jnp.float32),
        mesh=mesh,
        scratch_types=[
            pltpu.VMEM((C_PER_W, H), jnp.float32),
            pltpu.VMEM((H * W,), jnp.float32),
            pltpu.VMEM((H * W,), jnp.float32),
            pltpu.SemaphoreType.DMA,
            pltpu.SemaphoreType.DMA,
        ],
        compiler_params=pltpu.CompilerParams(
            use_tc_tiling_on_sc=False, needs_layout_passes=False),
    )
    def body(t_hbm, out_hbm, tv, buf0, buf1, sem0, sem1):
        wid = lax.axis_index("s") * NC + lax.axis_index("c")
        base_c = wid * C_PER_W
        is_col = wid < NWORKERS // 2

        pltpu.sync_copy(t_hbm.at[pl.ds(base_c, C_PER_W)], tv)

        bufs = (buf0, buf1)
        sems = (sem0, sem1)
        pending = {0: [], 1: []}

        for ci in range(C_PER_W):
            k = ci % 2
            buf = bufs[k]
            for hcopy in pending[k]:
                hcopy.wait()
            pending[k] = []

            @pl.when(is_col)
            def _(buf=buf, ci=ci):
                vs = [tv[ci, pl.ds(j * LANES, LANES)] for j in range(n_full)]
                vtail = tv[ci, pl.ds(tail_off, LANES)]

                def row_body(y, carry):
                    base = y * W
                    for j in range(n_full):
                        buf[pl.ds(base + j * LANES, LANES)] = vs[j]
                    buf[pl.ds(base + tail_off, LANES)] = vtail
                    return carry

                lax.fori_loop(0, H, row_body, 0)

            @pl.when(jnp.logical_not(is_col))
            def _(buf=buf, ci=ci):
                row_idx = jnp.full((LANES,), ci, jnp.int32)

                def row_body(y, carry):
                    v = plsc.load_gather(
                        tv, [row_idx, jnp.full((LANES,), y, jnp.int32)])
                    base = y * W
                    for j in range(n_full):
                        buf[pl.ds(base + j * LANES, LANES)] = v
                    buf[pl.ds(base + tail_off, LANES)] = v
                    return carry

                lax.fori_loop(0, H, row_body, 0)

            cch = base_c + ci
            for b in range(bs):
                pending[k].append(
                    pltpu.async_copy(buf, out_hbm.at[b, cch], sems[k]))

        for k in (0, 1):
            for hcopy in pending[k]:
                hcopy.wait()

    return body


def kernel(mask, row_weight, col_weight):
    bs = mask.shape[0]
    t = jnp.concatenate([col_weight.T, row_weight.T], axis=0)
    out = _make_kernel(bs)(t)
    return out.reshape(bs, C_TOTAL, H, W)


# speedup vs baseline: 1.9878x; 1.1475x over previous
import functools

import jax
import jax.numpy as jnp
from jax import lax
from jax.experimental import pallas as pl
from jax.experimental.pallas import tpu as pltpu
from jax.experimental.pallas import tpu_sc as plsc

NUM_FEATS = 128
H = 200
W = 200
NC = 2
NS = 16
LANES = 16
NWORKERS = NC * NS
C_TOTAL = 2 * NUM_FEATS
C_PER_W = C_TOTAL // NWORKERS


def _make_kernel(bs):
    mesh = plsc.VectorSubcoreMesh(
        core_axis_name="c", subcore_axis_name="s",
        num_cores=NC, num_subcores=NS)

    n_full = W // LANES
    tail_off = W - LANES

    @functools.partial(
        pl.kernel,
        out_type=jax.ShapeDtypeStruct((bs, C_TOTAL, H, W), jnp.float32),
        mesh=mesh,
        scratch_types=[
            pltpu.VMEM((C_PER_W, LANES * H), jnp.float32),
            pltpu.VMEM((H, W), jnp.float32),
            pltpu.VMEM((H, W), jnp.float32),
            pltpu.SemaphoreType.DMA,
            pltpu.SemaphoreType.DMA,
        ],
    )
    def body(u_hbm, out_hbm, uv, buf0, buf1, sem0, sem1):
        wid = lax.axis_index("s") * NC + lax.axis_index("c")
        base_c = wid * C_PER_W
        is_col = wid < NWORKERS // 2

        pltpu.sync_copy(u_hbm.at[pl.ds(base_c, C_PER_W)], uv)

        bufs = (buf0, buf1)
        sems = (sem0, sem1)
        pending = {0: [], 1: []}

        for ci in range(C_PER_W):
            k = ci % 2
            buf = bufs[k]
            for hcopy in pending[k]:
                hcopy.wait()
            pending[k] = []

            @pl.when(is_col)
            def _(buf=buf, ci=ci):
                vs = [uv[ci, pl.ds(j * LANES, LANES)] for j in range(n_full)]
                vtail = uv[ci, pl.ds(tail_off, LANES)]

                def row_body(y, carry):
                    for j in range(n_full):
                        buf[y, pl.ds(j * LANES, LANES)] = vs[j]
                    buf[y, pl.ds(tail_off, LANES)] = vtail
                    return carry

                lax.fori_loop(0, H, row_body, 0)

            @pl.when(jnp.logical_not(is_col))
            def _(buf=buf, ci=ci):
                def row_body(y, carry):
                    v = uv[ci, pl.ds(y * LANES, LANES)]
                    for j in range(n_full):
                        buf[y, pl.ds(j * LANES, LANES)] = v
                    buf[y, pl.ds(tail_off, LANES)] = v
                    return carry

                lax.fori_loop(0, H, row_body, 0)

            cch = base_c + ci
            for b in range(bs):
                pending[k].append(
                    pltpu.async_copy(buf, out_hbm.at[b, cch], sems[k]))

        for k in (0, 1):
            for hcopy in pending[k]:
                hcopy.wait()

    return body


def kernel(mask, row_weight, col_weight):
    bs = mask.shape[0]
    u_col = jnp.pad(col_weight.T, ((0, 0), (0, (LANES - 1) * H)))
    u_row = jnp.broadcast_to(
        row_weight.T[:, :, None], (NUM_FEATS, H, LANES)
    ).reshape(NUM_FEATS, LANES * H)
    u = jnp.concatenate([u_col, u_row], axis=0)
    return _make_kernel(bs)(u)


# speedup vs baseline: 1.9906x; 1.0014x over previous
import functools

import jax
import jax.numpy as jnp
from jax import lax
from jax.experimental import pallas as pl
from jax.experimental.pallas import tpu as pltpu
from jax.experimental.pallas import tpu_sc as plsc

NUM_FEATS = 128
H = 200
W = 200
NC = 2
NS = 16
LANES = 16
NWORKERS = NC * NS
C_TOTAL = 2 * NUM_FEATS
C_PER_W = C_TOTAL // NWORKERS


def _make_kernel(bs):
    mesh = plsc.VectorSubcoreMesh(
        core_axis_name="c", subcore_axis_name="s",
        num_cores=NC, num_subcores=NS)

    n_full = W // LANES
    tail_off = W - LANES

    @functools.partial(
        pl.kernel,
        out_type=jax.ShapeDtypeStruct((bs, C_TOTAL, H, W), jnp.float32),
        mesh=mesh,
        scratch_types=[
            pltpu.VMEM((C_PER_W, LANES * H), jnp.float32),
            pltpu.VMEM((H, W), jnp.float32),
            pltpu.VMEM((H, W), jnp.float32),
            pltpu.SemaphoreType.DMA,
            pltpu.SemaphoreType.DMA,
        ],
        compiler_params=pltpu.CompilerParams(use_tc_tiling_on_sc=True),
    )
    def body(u_hbm, out_hbm, uv, buf0, buf1, sem0, sem1):
        wid = lax.axis_index("s") * NC + lax.axis_index("c")
        base_c = wid * C_PER_W
        is_col = wid < NWORKERS // 2

        pltpu.sync_copy(u_hbm.at[pl.ds(base_c, C_PER_W)], uv)

        bufs = (buf0, buf1)
        sems = (sem0, sem1)
        pending = {0: [], 1: []}

        for ci in range(C_PER_W):
            k = ci % 2
            buf = bufs[k]
            for hcopy in pending[k]:
                hcopy.wait()
            pending[k] = []

            @pl.when(is_col)
            def _(buf=buf, ci=ci):
                vs = [uv[ci, pl.ds(j * LANES, LANES)] for j in range(n_full)]
                vtail = uv[ci, pl.ds(tail_off, LANES)]

                def row_body(y, carry):
                    for j in range(n_full):
                        buf[y, pl.ds(j * LANES, LANES)] = vs[j]
                    buf[y, pl.ds(tail_off, LANES)] = vtail
                    return carry

                lax.fori_loop(0, H, row_body, 0)

            @pl.when(jnp.logical_not(is_col))
            def _(buf=buf, ci=ci):
                def row_body(y, carry):
                    v = uv[ci, pl.ds(y * LANES, LANES)]
                    for j in range(n_full):
                        buf[y, pl.ds(j * LANES, LANES)] = v
                    buf[y, pl.ds(tail_off, LANES)] = v
                    return carry

                lax.fori_loop(0, H, row_body, 0)

            cch = base_c + ci
            for b in range(bs):
                pending[k].append(
                    pltpu.async_copy(buf, out_hbm.at[b, cch], sems[k]))

        for k in (0, 1):
            for hcopy in pending[k]:
                hcopy.wait()

    return body


def kernel(mask, row_weight, col_weight):
    bs = mask.shape[0]
    u_col = jnp.pad(col_weight.T, ((0, 0), (0, (LANES - 1) * H)))
    u_row = jnp.broadcast_to(
        row_weight.T[:, :, None], (NUM_FEATS, H, LANES)
    ).reshape(NUM_FEATS, LANES * H)
    u = jnp.concatenate([u_col, u_row], axis=0)
    return _make_kernel(bs)(u)


# speedup vs baseline: 7.0927x; 3.5631x over previous
import functools

import jax
import jax.numpy as jnp
from jax import lax
from jax.experimental import pallas as pl
from jax.experimental.pallas import tpu as pltpu
from jax.experimental.pallas import tpu_sc as plsc

NUM_FEATS = 128
H = 200
W = 200
NC = 2
NS = 16
LANES = 16
NWORKERS = NC * NS
C_TOTAL = 2 * NUM_FEATS
Y_PER_W = 7


def _make_kernel(bs):
    mesh = plsc.VectorSubcoreMesh(
        core_axis_name="c", subcore_axis_name="s",
        num_cores=NC, num_subcores=NS)

    n_half = NUM_FEATS // LANES

    @functools.partial(
        pl.kernel,
        out_type=jax.ShapeDtypeStruct((bs, H, W, C_TOTAL), jnp.float32),
        mesh=mesh,
        scratch_types=[
            pltpu.VMEM((16, NUM_FEATS), jnp.float32),
            pltpu.VMEM((W, C_TOTAL), jnp.float32),
            pltpu.VMEM((W, C_TOTAL), jnp.float32),
            pltpu.SemaphoreType.DMA,
            pltpu.SemaphoreType.DMA,
        ],
        compiler_params=pltpu.CompilerParams(use_tc_tiling_on_sc=True),
    )
    def body(rw_hbm, cw_hbm, out_hbm, rv, buf0, buf1, sem0, sem1):
        wid = lax.axis_index("s") * NC + lax.axis_index("c")
        base_y = wid * H // NWORKERS

        base8 = jnp.minimum((base_y // 8) * 8, H - 16)
        off = base_y - base8
        pltpu.sync_copy(rw_hbm.at[pl.ds(base8, 16)], rv)
        pltpu.sync_copy(cw_hbm, buf0.at[:, pl.ds(0, NUM_FEATS)])
        pltpu.sync_copy(cw_hbm, buf1.at[:, pl.ds(0, NUM_FEATS)])

        bufs = (buf0, buf1)
        sems = (sem0, sem1)
        pending = {0: [], 1: []}

        for t in range(Y_PER_W):
            k = t % 2
            buf = bufs[k]
            for hcopy in pending[k]:
                hcopy.wait()
            pending[k] = []

            vs = [rv[off + t, pl.ds(j * LANES, LANES)] for j in range(n_half)]

            def row_body(x, carry, buf=buf, vs=vs):
                for j in range(n_half):
                    buf[x, pl.ds(NUM_FEATS + j * LANES, LANES)] = vs[j]
                return carry

            lax.fori_loop(0, W, row_body, 0)

            yy = base_y + t
            for b in range(bs):
                pending[k].append(
                    pltpu.async_copy(buf, out_hbm.at[b, yy], sems[k]))

        for k in (0, 1):
            for hcopy in pending[k]:
                hcopy.wait()

    return body


def kernel(mask, row_weight, col_weight):
    bs = mask.shape[0]
    out = _make_kernel(bs)(row_weight, col_weight)
    return jnp.transpose(out, (0, 3, 1, 2))


# speedup vs baseline: 7.2312x; 1.0195x over previous
import functools

import jax
import jax.numpy as jnp
from jax import lax
from jax.experimental import pallas as pl
from jax.experimental.pallas import tpu as pltpu
from jax.experimental.pallas import tpu_sc as plsc

NUM_FEATS = 128
H = 200
W = 200
NC = 2
NS = 16
LANES = 16
NWORKERS = NC * NS
C_TOTAL = 2 * NUM_FEATS
Y_PER_W = 7


def _make_kernel(bs):
    mesh = plsc.VectorSubcoreMesh(
        core_axis_name="c", subcore_axis_name="s",
        num_cores=NC, num_subcores=NS)

    n_half = NUM_FEATS // LANES

    @functools.partial(
        pl.kernel,
        out_type=jax.ShapeDtypeStruct((bs, H, W, C_TOTAL), jnp.float32),
        mesh=mesh,
        scratch_types=[
            pltpu.VMEM((16, NUM_FEATS), jnp.float32),
            pltpu.VMEM((W, C_TOTAL), jnp.float32),
            pltpu.VMEM((W, C_TOTAL), jnp.float32),
            pltpu.SemaphoreType.DMA,
            pltpu.SemaphoreType.DMA,
        ],
        compiler_params=pltpu.CompilerParams(use_tc_tiling_on_sc=True),
    )
    def body(rw_hbm, cw_hbm, out_hbm, rv, buf0, buf1, sem0, sem1):
        wid = lax.axis_index("s") * NC + lax.axis_index("c")
        base_y = 6 * wid + jnp.minimum(wid, 8)

        base8 = jnp.minimum((base_y // 8) * 8, H - 16)
        off = base_y - base8
        pltpu.sync_copy(rw_hbm.at[pl.ds(base8, 16)], rv)
        pltpu.sync_copy(cw_hbm, buf0.at[:, pl.ds(0, NUM_FEATS)])
        pltpu.sync_copy(cw_hbm, buf1.at[:, pl.ds(0, NUM_FEATS)])

        bufs = (buf0, buf1)
        sems = (sem0, sem1)
        pending = {0: [], 1: []}

        def do_slab(t, buf, sem, plist):
            vs = [rv[off + t, pl.ds(j * LANES, LANES)] for j in range(n_half)]

            def row_body(x, carry):
                for j in range(n_half):
                    buf[x, pl.ds(NUM_FEATS + j * LANES, LANES)] = vs[j]
                return carry

            lax.fori_loop(0, W, row_body, 0)

            yy = base_y + t
            for b in range(bs):
                plist.append(pltpu.async_copy(buf, out_hbm.at[b, yy], sem))

        for t in range(Y_PER_W - 1):
            k = t % 2
            for hcopy in pending[k]:
                hcopy.wait()
            pending[k] = []
            do_slab(t, bufs[k], sems[k], pending[k])

        @pl.when(wid < 8)
        def _():
            for hcopy in pending[0]:
                hcopy.wait()
            extra = []
            do_slab(Y_PER_W - 1, bufs[0], sems[0], extra)

        for k in (0, 1):
            for hcopy in pending[k]:
                hcopy.wait()

    return body


def kernel(mask, row_weight, col_weight):
    bs = mask.shape[0]
    out = _make_kernel(bs)(row_weight, col_weight)
    return jnp.transpose(out, (0, 3, 1, 2))


# speedup vs baseline: 7.7396x; 1.0703x over previous
import functools

import jax
import jax.numpy as jnp
from jax import lax
from jax.experimental import pallas as pl
from jax.experimental.pallas import tpu as pltpu
from jax.experimental.pallas import tpu_sc as plsc

NUM_FEATS = 128
H = 200
W = 200
NC = 2
NS = 16
LANES = 16
NWORKERS = NC * NS
C_TOTAL = 2 * NUM_FEATS


def _make_kernel(bs):
    mesh = plsc.VectorSubcoreMesh(
        core_axis_name="c", subcore_axis_name="s",
        num_cores=NC, num_subcores=NS)

    n_half = NUM_FEATS // LANES
    y_groups = NWORKERS // bs
    slabs_per_w = H // y_groups
    n_pairs = slabs_per_w // 2

    @functools.partial(
        pl.kernel,
        out_type=jax.ShapeDtypeStruct((bs, H, W, C_TOTAL), jnp.float32),
        mesh=mesh,
        scratch_types=[
            pltpu.VMEM((H, NUM_FEATS), jnp.float32),
            pltpu.VMEM((W, C_TOTAL), jnp.float32),
            pltpu.VMEM((W, C_TOTAL), jnp.float32),
            pltpu.SemaphoreType.DMA,
            pltpu.SemaphoreType.DMA,
        ],
        compiler_params=pltpu.CompilerParams(use_tc_tiling_on_sc=True),
    )
    def body(rw_hbm, cw_hbm, out_hbm, rv, buf0, buf1, sem0, sem1):
        wid = lax.axis_index("s") * NC + lax.axis_index("c")
        bb = wid % bs
        y0 = wid // bs

        pltpu.sync_copy(rw_hbm, rv)
        pltpu.sync_copy(cw_hbm, buf0.at[:, pl.ds(0, NUM_FEATS)])
        pltpu.sync_copy(cw_hbm, buf1.at[:, pl.ds(0, NUM_FEATS)])

        def build(y, buf):
            vs = [rv[y, pl.ds(j * LANES, LANES)] for j in range(n_half)]

            def row_body(x, carry):
                for j in range(n_half):
                    buf[x, pl.ds(NUM_FEATS + j * LANES, LANES)] = vs[j]
                return carry

            lax.fori_loop(0, W, row_body, 0)

        def fire(y, buf, sem):
            pltpu.async_copy(buf, out_hbm.at[bb, y], sem)

        def drain(buf, sem):
            pltpu.make_async_copy(buf, out_hbm.at[bb, y0], sem).wait()

        build(y0, buf0)
        fire(y0, buf0, sem0)
        build(y0 + y_groups, buf1)
        fire(y0 + y_groups, buf1, sem1)

        def pair_body(i, carry):
            y_a = y0 + y_groups * (2 * i)
            drain(buf0, sem0)
            build(y_a, buf0)
            fire(y_a, buf0, sem0)
            y_b = y_a + y_groups
            drain(buf1, sem1)
            build(y_b, buf1)
            fire(y_b, buf1, sem1)
            return carry

        lax.fori_loop(1, n_pairs, pair_body, 0)

        drain(buf0, sem0)
        drain(buf1, sem1)

    return body


def kernel(mask, row_weight, col_weight):
    bs = mask.shape[0]
    out = _make_kernel(bs)(row_weight, col_weight)
    return jnp.transpose(out, (0, 3, 1, 2))
